# baseline (device time: 9463 ns/iter reference)
import jax
import jax.numpy as jnp
from jax import lax
from jax.experimental import pallas as pl
from jax.experimental.pallas import tpu as pltpu


def kernel(x, gamma, beta):
    m, n = x.shape
    n_global = 2 * n
    eps = 1e-5

    def body(x_ref, g_ref, b_ref, out_ref,
             stats_send, stats_recv, send_sem, recv_sem):
        my_x = lax.axis_index("x")
        my_y = lax.axis_index("y")
        peer = (my_x, 1 - my_y)

        barrier_sem = pltpu.get_barrier_semaphore()
        pl.semaphore_signal(
            barrier_sem, inc=1,
            device_id=peer, device_id_type=pl.DeviceIdType.MESH,
        )
        pl.semaphore_wait(barrier_sem, 1)

        xv = x_ref[:, :]
        stats_send[:, 0:1] = jnp.sum(xv, axis=1, keepdims=True)
        stats_send[:, 1:2] = jnp.sum(xv * xv, axis=1, keepdims=True)

        rdma = pltpu.make_async_remote_copy(
            src_ref=stats_send,
            dst_ref=stats_recv,
            send_sem=send_sem,
            recv_sem=recv_sem,
            device_id=peer,
            device_id_type=pl.DeviceIdType.MESH,
        )
        rdma.start()
        rdma.wait()

        total = stats_send[:, :] + stats_recv[:, :]
        mean = total[:, 0:1] / n_global
        var = total[:, 1:2] / n_global - mean * mean
        inv = lax.rsqrt(var + eps)
        out_ref[:, :] = (xv - mean) * inv * g_ref[:, :] + b_ref[:, :]

    return pl.pallas_call(
        body,
        out_shape=jax.ShapeDtypeStruct((m, n), jnp.float32),
        in_specs=[
            pl.BlockSpec(memory_space=pltpu.VMEM),
            pl.BlockSpec(memory_space=pltpu.VMEM),
            pl.BlockSpec(memory_space=pltpu.VMEM),
        ],
        out_specs=pl.BlockSpec(memory_space=pltpu.VMEM),
        scratch_shapes=[
            pltpu.VMEM((m, 2), jnp.float32),
            pltpu.VMEM((m, 2), jnp.float32),
            pltpu.SemaphoreType.DMA,
            pltpu.SemaphoreType.DMA,
        ],
        compiler_params=pltpu.CompilerParams(collective_id=0),
    )(x, gamma.reshape(1, n), beta.reshape(1, n))
